# Initial kernel scaffold; baseline (speedup 1.0000x reference)
import jax
import jax.numpy as jnp
from jax.experimental import pallas as pl


def kernel(month, day, weekday, hour, month_table, day_table, weekday_table, hour_table, W1, b1, W2, b2, W3, b3):
    raise NotImplementedError("write your pallas kernel here")


# speedup vs baseline: 11.0674x; 11.0674x over previous
import jax
import jax.numpy as jnp
from jax.experimental import pallas as pl

_B = 16384
_R = 2048
_NSTEP = _B // _R


def _body(m_ref, d_ref, w_ref, h_ref, tcat_ref, w1_ref, b1_ref, w2_ref,
          b2_ref, w3_ref, b3_ref, out_ref):
    f32 = jnp.float32
    m = m_ref[0]
    d = d_ref[0]
    w = w_ref[0]
    h = h_ref[0]
    iota = jax.lax.broadcasted_iota(jnp.int32, (128, _R), 0)
    hot = ((iota == m) | (iota == d + 13) | (iota == w + 45)
           | (iota == h + 52))
    mh = jnp.where(hot, f32(1.0), f32(0.0))

    fold = jnp.dot(tcat_ref[...], w1_ref[...],
                   preferred_element_type=f32)

    c00 = (((0,), (0,)), ((), ()))
    h1 = jax.lax.dot_general(fold, mh, c00,
                             preferred_element_type=f32)
    h1 = jnp.maximum(h1 + b1_ref[...], 0.0)
    h2 = jax.lax.dot_general(w2_ref[...], h1, c00,
                             preferred_element_type=f32)
    h2 = jnp.maximum(h2 + b2_ref[...], 0.0)
    o = jax.lax.dot_general(w3_ref[...], h2, c00,
                            preferred_element_type=f32)
    o = jnp.maximum(o + b3_ref[...], 0.0)
    out_ref[...] = o.reshape(1, 1, _R)


def kernel(month, day, weekday, hour, month_table, day_table, weekday_table,
           hour_table, W1, b1, W2, b2, W3, b3):
    i32 = jnp.int32
    f32 = jnp.float32
    m = month.astype(i32).reshape(_NSTEP, 1, _R)
    d = day.astype(i32).reshape(_NSTEP, 1, _R)
    w = weekday.astype(i32).reshape(_NSTEP, 1, _R)
    h = hour.astype(i32).reshape(_NSTEP, 1, _R)
    tcat = jnp.concatenate(
        [month_table, day_table, weekday_table, hour_table,
         jnp.zeros((52, 128), f32)], axis=0)
    b1c = b1.reshape(64, 1)
    b2c = b2.reshape(32, 1)
    b3c = b3.reshape(1, 1)

    idx_spec = pl.BlockSpec((1, 1, _R), lambda i: (i, 0, 0))
    full = lambda s: pl.BlockSpec(s, lambda i: tuple(0 for _ in s))
    out = pl.pallas_call(
        _body,
        grid=(_NSTEP,),
        in_specs=[idx_spec, idx_spec, idx_spec, idx_spec,
                  full((128, 128)), full((128, 64)), full((64, 1)),
                  full((64, 32)), full((32, 1)), full((32, 1)),
                  full((1, 1))],
        out_specs=pl.BlockSpec((1, 1, _R), lambda i: (i, 0, 0)),
        out_shape=jax.ShapeDtypeStruct((_NSTEP, 1, _R), f32),
    )(m, d, w, h, tcat, W1, b1c, W2, b2c, W3, b3c)
    return out.reshape(_B, 1)
